# initial kernel scaffold (unmeasured)
import functools

import jax
import jax.numpy as jnp
from jax import lax
from jax.experimental import pallas as pl
from jax.experimental.pallas import tpu as pltpu

N_DEV = 4
M_BLK = 1024
K = 4096
N = 8192
NB = 1024
N_STEPS = N // NB


def kernel(x, w_mat):
    def body(x_ref, w_ref, out_ref, xfull_ref, send_sems, recv_sems):
        step = pl.program_id(0)
        my = lax.axis_index("i")

        @pl.when(step == 0)
        def _comm():
            barrier_sem = pltpu.get_barrier_semaphore()
            for off in range(1, N_DEV):
                nbr = lax.rem(my + off, N_DEV)
                pl.semaphore_signal(
                    barrier_sem, inc=1,
                    device_id=(nbr,), device_id_type=pl.DeviceIdType.MESH,
                )
            pl.semaphore_wait(barrier_sem, N_DEV - 1)

            xfull_ref[:, pl.ds(my * M_BLK, M_BLK)] = (
                x_ref[pl.ds(my * M_BLK, M_BLK), :]
            )

            sends = []
            for off in range(1, N_DEV):
                dst = lax.rem(my + off, N_DEV)
                rdma = pltpu.make_async_remote_copy(
                    src_ref=x_ref.at[pl.ds(dst * M_BLK, M_BLK), :],
                    dst_ref=xfull_ref.at[:, pl.ds(my * M_BLK, M_BLK)],
                    send_sem=send_sems.at[off],
                    recv_sem=recv_sems.at[off],
                    device_id=(dst,),
                    device_id_type=pl.DeviceIdType.MESH,
                )
                rdma.start()
                sends.append(rdma)

            for off in range(1, N_DEV):
                src = lax.rem(my + (N_DEV - off), N_DEV)
                recv = pltpu.make_async_remote_copy(
                    src_ref=x_ref.at[pl.ds(0, M_BLK), :],
                    dst_ref=xfull_ref.at[:, pl.ds(src * M_BLK, M_BLK)],
                    send_sem=send_sems.at[0],
                    recv_sem=recv_sems.at[off],
                    device_id=(my,),
                    device_id_type=pl.DeviceIdType.MESH,
                )
                recv.wait_recv()
            for rdma in sends:
                rdma.wait_send()

        acc = jnp.dot(
            xfull_ref[:, :], w_ref[:, :], preferred_element_type=jnp.float32
        )
        out_ref[:, :] = jnp.maximum(acc, 0.0)

        @pl.when(step == N_STEPS - 1)
        def _exit_barrier():
            @functools.partial(
                pl.run_scoped, second_barrier=pltpu.SemaphoreType.REGULAR
            )
            def _(second_barrier):
                for off in range(1, N_DEV):
                    nbr = lax.rem(my + off, N_DEV)
                    pl.semaphore_signal(
                        second_barrier, inc=1,
                        device_id=(nbr,),
                        device_id_type=pl.DeviceIdType.MESH,
                    )
                pl.semaphore_wait(second_barrier, N_DEV - 1)

    return pl.pallas_call(
        body,
        grid=(N_STEPS,),
        in_specs=[
            pl.BlockSpec((K, M_BLK), lambda n: (0, 0)),
            pl.BlockSpec((K, NB), lambda n: (0, n)),
        ],
        out_specs=pl.BlockSpec((M_BLK, NB), lambda n: (0, n)),
        out_shape=jax.ShapeDtypeStruct((M_BLK, N), jnp.float32),
        scratch_shapes=[
            pltpu.VMEM((M_BLK, K), x.dtype),
            pltpu.SemaphoreType.DMA((N_DEV,)),
            pltpu.SemaphoreType.DMA((N_DEV,)),
        ],
        compiler_params=pltpu.CompilerParams(
            collective_id=0, dimension_semantics=("arbitrary",)
        ),
    )(x, w_mat)


# baseline (device time: 166211 ns/iter reference)
import functools

import jax
import jax.numpy as jnp
from jax import lax
from jax.experimental import pallas as pl
from jax.experimental.pallas import tpu as pltpu

N_DEV = 4
M_BLK = 1024
K = 4096
N = 8192
NB = 1024
N_STEPS = N // NB


def kernel(x, w_mat):
    x = x.astype(jnp.bfloat16)

    def body(x_hbm, w_ref, out_ref, xfull_ref, send_sems, recv_sems):
        step = pl.program_id(0)
        my = lax.axis_index("i")

        @pl.when(step == 0)
        def _comm():
            barrier_sem = pltpu.get_barrier_semaphore()
            for off in range(1, N_DEV):
                nbr = lax.rem(my + off, N_DEV)
                pl.semaphore_signal(
                    barrier_sem, inc=1,
                    device_id=(nbr,), device_id_type=pl.DeviceIdType.MESH,
                )
            pl.semaphore_wait(barrier_sem, N_DEV - 1)

            local = pltpu.make_async_copy(
                x_hbm.at[pl.ds(my * M_BLK, M_BLK), :],
                xfull_ref.at[:, pl.ds(my * M_BLK, M_BLK)],
                send_sems.at[0],
            )
            local.start()

            sends = []
            for off in range(1, N_DEV):
                dst = lax.rem(my + off, N_DEV)
                rdma = pltpu.make_async_remote_copy(
                    src_ref=x_hbm.at[pl.ds(dst * M_BLK, M_BLK), :],
                    dst_ref=xfull_ref.at[:, pl.ds(my * M_BLK, M_BLK)],
                    send_sem=send_sems.at[off],
                    recv_sem=recv_sems.at[off],
                    device_id=(dst,),
                    device_id_type=pl.DeviceIdType.MESH,
                )
                rdma.start()
                sends.append(rdma)

            local.wait()

            for off in range(1, N_DEV):
                src = lax.rem(my + (N_DEV - off), N_DEV)
                recv = pltpu.make_async_remote_copy(
                    src_ref=x_hbm.at[pl.ds(0, M_BLK), :],
                    dst_ref=xfull_ref.at[:, pl.ds(src * M_BLK, M_BLK)],
                    send_sem=send_sems.at[0],
                    recv_sem=recv_sems.at[off],
                    device_id=(my,),
                    device_id_type=pl.DeviceIdType.MESH,
                )
                recv.wait_recv()
            for rdma in sends:
                rdma.wait_send()

        acc = jnp.dot(
            xfull_ref[:, :],
            w_ref[:, :].astype(jnp.bfloat16),
            preferred_element_type=jnp.float32,
        )
        out_ref[:, :] = jnp.maximum(acc, 0.0)

        @pl.when(step == N_STEPS - 1)
        def _exit_barrier():
            @functools.partial(
                pl.run_scoped, second_barrier=pltpu.SemaphoreType.REGULAR
            )
            def _(second_barrier):
                for off in range(1, N_DEV):
                    nbr = lax.rem(my + off, N_DEV)
                    pl.semaphore_signal(
                        second_barrier, inc=1,
                        device_id=(nbr,),
                        device_id_type=pl.DeviceIdType.MESH,
                    )
                pl.semaphore_wait(second_barrier, N_DEV - 1)

    return pl.pallas_call(
        body,
        grid=(N_STEPS,),
        in_specs=[
            pl.BlockSpec(memory_space=pltpu.MemorySpace.HBM),
            pl.BlockSpec((K, NB), lambda n: (0, n)),
        ],
        out_specs=pl.BlockSpec((M_BLK, NB), lambda n: (0, n)),
        out_shape=jax.ShapeDtypeStruct((M_BLK, N), jnp.float32),
        scratch_shapes=[
            pltpu.VMEM((M_BLK, K), jnp.bfloat16),
            pltpu.SemaphoreType.DMA((N_DEV,)),
            pltpu.SemaphoreType.DMA((N_DEV,)),
        ],
        compiler_params=pltpu.CompilerParams(
            collective_id=0,
            dimension_semantics=("arbitrary",),
            vmem_limit_bytes=60 * 1024 * 1024,
        ),
    )(x, w_mat)
